# baseline (device time: 66701 ns/iter reference)
import jax
import jax.numpy as jnp
from jax import lax
from jax.experimental import pallas as pl
from jax.experimental.pallas import tpu as pltpu

N_DEV = 16

RING = [0, 1, 5, 9, 13, 14, 10, 6, 2, 3, 7, 11, 15, 12, 8, 4]
POS = [0] * N_DEV
NEXT = [0] * N_DEV
PREV = [0] * N_DEV
for _j, _m in enumerate(RING):
    POS[_m] = _j
    NEXT[_m] = RING[(_j + 1) % N_DEV]
    PREV[_m] = RING[(_j - 1) % N_DEV]

N_PIECE = 2
N_SLOT = 3
N_HOP = 8

LAST_CW = [N_HOP - 1, N_HOP - 2]
LAST_CCW = [N_HOP - 2, N_HOP - 1]

ORG_CW = [
    [RING[(POS[m] - h - 1) % N_DEV] for m in range(N_DEV)]
    for h in range(N_HOP)
]
ORG_CCW = [
    [RING[(POS[m] + h + 1) % N_DEV] for m in range(N_DEV)]
    for h in range(N_HOP)
]


def _lut(idx, table):
    v = jnp.int32(table[0])
    for j in range(1, len(table)):
        v = jnp.where(idx == j, jnp.int32(table[j]), v)
    return v


def kernel(A, B):
    m_per, k = A.shape
    _, n = B.shape
    ph = m_per // N_PIECE

    n_st = 4

    def body(a_ref, b_ref, out_ref, cw_ref, ccw_ref,
             send_cw, recv_cw, send_ccw, recv_ccw, st_buf, st_sems):
        my = lax.axis_index("i")
        nxt = _lut(my, NEXT)
        prv = _lut(my, PREV)

        barrier_sem = pltpu.get_barrier_semaphore()
        for nbr in [prv, nxt]:
            pl.semaphore_signal(
                barrier_sem, inc=1,
                device_id=(nbr,), device_id_type=pl.DeviceIdType.MESH,
            )
        pl.semaphore_wait(barrier_sem, 2)

        def mk(buf, s_sems, r_sems, dev, h, p):
            return pltpu.make_async_remote_copy(
                src_ref=buf.at[h % N_SLOT, p],
                dst_ref=buf.at[(h + 1) % N_SLOT, p],
                send_sem=s_sems.at[h % N_SLOT, p],
                recv_sem=r_sems.at[(h + 1) % N_SLOT, p],
                device_id=(dev,),
                device_id_type=pl.DeviceIdType.MESH,
            )

        def mk_cw(h, p):
            return mk(cw_ref, send_cw, recv_cw, nxt, h, p)

        def mk_ccw(h, p):
            return mk(ccw_ref, send_ccw, recv_ccw, prv, h, p)

        cw_d = {}
        ccw_d = {}
        waited = set()

        a_pc = []
        for p in range(N_PIECE):
            ap = a_ref[p * ph:(p + 1) * ph, :].astype(jnp.bfloat16)
            a_pc.append(ap)
            cw_ref[0, p] = ap
            ccw_ref[0, p] = ap
            cw_d[(0, p)] = mk_cw(0, p)
            cw_d[(0, p)].start()
            ccw_d[(0, p)] = mk_ccw(0, p)
            ccw_d[(0, p)].start()

        b_bf = b_ref[...].astype(jnp.bfloat16)

        st_state = {"cnt": 0, "pending": {}}

        def emit(c_bf16, row_start):
            cnt = st_state["cnt"]
            slot = cnt % n_st
            if cnt >= n_st:
                st_state["pending"][slot].wait()
            st_buf[slot] = c_bf16
            cp = pltpu.make_async_copy(
                st_buf.at[slot],
                out_ref.at[pl.ds(row_start, ph), :],
                st_sems.at[slot],
            )
            cp.start()
            st_state["pending"][slot] = cp
            st_state["cnt"] = cnt + 1

        def tile(args):
            buf, h, p, org_table = args
            slot = (h + 1) % N_SLOT
            org = _lut(my, org_table[h])
            c = jnp.dot(buf[slot, p], b_bf, preferred_element_type=jnp.float32)
            emit(c.astype(jnp.bfloat16), org * m_per + p * ph)

        for p in range(N_PIECE):
            c = jnp.dot(a_pc[p], b_bf, preferred_element_type=jnp.float32)
            emit(c.astype(jnp.bfloat16), my * m_per + p * ph)

        def recv_fwd(d, mk_fn, last, h, p):
            if h > last:
                return
            d[(h, p)].wait_recv()
            if h + 1 <= last:
                if h - 2 >= 0:
                    d[(h - 2, p)].wait_send()
                    waited.add((id(d), h - 2, p))
                d[(h + 1, p)] = mk_fn(h + 1, p)
                d[(h + 1, p)].start()

        def hop_tiles(h):
            ts = []
            if h <= LAST_CW[0]:
                ts.append((cw_ref, h, 0, ORG_CW))
            if h <= LAST_CCW[0]:
                ts.append((ccw_ref, h, 0, ORG_CCW))
            if h <= LAST_CW[1]:
                ts.append((cw_ref, h, 1, ORG_CW))
            if h <= LAST_CCW[1]:
                ts.append((ccw_ref, h, 1, ORG_CCW))
            return ts

        for h in range(N_HOP):
            lag = hop_tiles(h - 1) if h > 0 else []
            recv_fwd(cw_d, mk_cw, LAST_CW[0], h, 0)
            recv_fwd(ccw_d, mk_ccw, LAST_CCW[0], h, 0)
            for t in lag[:2]:
                tile(t)
            recv_fwd(cw_d, mk_cw, LAST_CW[1], h, 1)
            recv_fwd(ccw_d, mk_ccw, LAST_CCW[1], h, 1)
            for t in lag[2:]:
                tile(t)

        for t in hop_tiles(N_HOP - 1):
            tile(t)
        for cp in st_state["pending"].values():
            cp.wait()
        for d in (cw_d, ccw_d):
            for (hh, pp), desc in d.items():
                if (id(d), hh, pp) not in waited:
                    desc.wait_send()

    return pl.pallas_call(
        body,
        out_shape=jax.ShapeDtypeStruct((N_DEV * m_per, n), jnp.bfloat16),
        in_specs=[
            pl.BlockSpec(memory_space=pltpu.VMEM),
            pl.BlockSpec(memory_space=pltpu.VMEM),
        ],
        out_specs=pl.BlockSpec(memory_space=pl.ANY),
        scratch_shapes=[
            pltpu.VMEM((N_SLOT, N_PIECE, ph, k), jnp.bfloat16),
            pltpu.VMEM((N_SLOT, N_PIECE, ph, k), jnp.bfloat16),
            pltpu.SemaphoreType.DMA((N_SLOT, N_PIECE)),
            pltpu.SemaphoreType.DMA((N_SLOT, N_PIECE)),
            pltpu.SemaphoreType.DMA((N_SLOT, N_PIECE)),
            pltpu.SemaphoreType.DMA((N_SLOT, N_PIECE)),
            pltpu.VMEM((n_st, ph, n), jnp.bfloat16),
            pltpu.SemaphoreType.DMA((n_st,)),
        ],
        compiler_params=pltpu.CompilerParams(collective_id=0),
    )(A, B)


# device time: 64013 ns/iter; 1.0420x vs baseline; 1.0420x over previous
import jax
import jax.numpy as jnp
from jax import lax
from jax.experimental import pallas as pl
from jax.experimental.pallas import tpu as pltpu

N_DEV = 16

RING = [0, 1, 5, 9, 13, 14, 10, 6, 2, 3, 7, 11, 15, 12, 8, 4]
POS = [0] * N_DEV
NEXT = [0] * N_DEV
PREV = [0] * N_DEV
for _j, _m in enumerate(RING):
    POS[_m] = _j
    NEXT[_m] = RING[(_j + 1) % N_DEV]
    PREV[_m] = RING[(_j - 1) % N_DEV]

N_PIECE = 2
N_SLOT = 3
N_HOP = 8

LAST_CW = [N_HOP - 1, N_HOP - 2]
LAST_CCW = [N_HOP - 2, N_HOP - 1]

ORG_CW = [
    [RING[(POS[m] - h - 1) % N_DEV] for m in range(N_DEV)]
    for h in range(N_HOP)
]
ORG_CCW = [
    [RING[(POS[m] + h + 1) % N_DEV] for m in range(N_DEV)]
    for h in range(N_HOP)
]


def _lut(idx, table):
    v = jnp.int32(table[0])
    for j in range(1, len(table)):
        v = jnp.where(idx == j, jnp.int32(table[j]), v)
    return v


def kernel(A, B):
    m_per, k = A.shape
    _, n = B.shape
    ph = m_per // N_PIECE

    def body(a_ref, b_ref, out_ref, cw_ref, ccw_ref,
             send_cw, recv_cw, send_ccw, recv_ccw):
        my = lax.axis_index("i")
        nxt = _lut(my, NEXT)
        prv = _lut(my, PREV)

        barrier_sem = pltpu.get_barrier_semaphore()
        for nbr in [prv, nxt]:
            pl.semaphore_signal(
                barrier_sem, inc=1,
                device_id=(nbr,), device_id_type=pl.DeviceIdType.MESH,
            )
        pl.semaphore_wait(barrier_sem, 2)

        def mk(buf, s_sems, r_sems, dev, h, p):
            return pltpu.make_async_remote_copy(
                src_ref=buf.at[h % N_SLOT, p],
                dst_ref=buf.at[(h + 1) % N_SLOT, p],
                send_sem=s_sems.at[h % N_SLOT, p],
                recv_sem=r_sems.at[(h + 1) % N_SLOT, p],
                device_id=(dev,),
                device_id_type=pl.DeviceIdType.MESH,
            )

        def mk_cw(h, p):
            return mk(cw_ref, send_cw, recv_cw, nxt, h, p)

        def mk_ccw(h, p):
            return mk(ccw_ref, send_ccw, recv_ccw, prv, h, p)

        cw_d = {}
        ccw_d = {}
        waited = set()

        a_pc = []
        for p in range(N_PIECE):
            ap = a_ref[p * ph:(p + 1) * ph, :].astype(jnp.bfloat16)
            a_pc.append(ap)
            cw_ref[0, p] = ap
            ccw_ref[0, p] = ap
            cw_d[(0, p)] = mk_cw(0, p)
            cw_d[(0, p)].start()
            ccw_d[(0, p)] = mk_ccw(0, p)
            ccw_d[(0, p)].start()

        b_bf = b_ref[...].astype(jnp.bfloat16)

        def tile(buf, h, p, org_table):
            slot = (h + 1) % N_SLOT
            org = _lut(my, org_table[h])
            c = jnp.dot(buf[slot, p], b_bf, preferred_element_type=jnp.float32)
            out_ref[pl.ds(org * m_per + p * ph, ph), :] = c.astype(jnp.bfloat16)

        for p in range(N_PIECE):
            c = jnp.dot(a_pc[p], b_bf, preferred_element_type=jnp.float32)
            out_ref[pl.ds(my * m_per + p * ph, ph), :] = c.astype(jnp.bfloat16)

        def recv_fwd(d, mk_fn, last, h, p):
            d[(h, p)].wait_recv()
            if h + 1 <= last:
                if h - 2 >= 0:
                    d[(h - 2, p)].wait_send()
                    waited.add((id(d), h - 2, p))
                d[(h + 1, p)] = mk_fn(h + 1, p)
                d[(h + 1, p)].start()

        for h in range(N_HOP - 1):
            recv_fwd(cw_d, mk_cw, LAST_CW[0], h, 0)
            recv_fwd(ccw_d, mk_ccw, LAST_CCW[0], h, 0)
            tile(cw_ref, h, 0, ORG_CW)
            recv_fwd(cw_d, mk_cw, LAST_CW[1], h, 1)
            recv_fwd(ccw_d, mk_ccw, LAST_CCW[1], h, 1)
            tile(ccw_ref, h, 0, ORG_CCW)
            tile(cw_ref, h, 1, ORG_CW)
            tile(ccw_ref, h, 1, ORG_CCW)

        h = N_HOP - 1
        cw_d[(h, 0)].wait_recv()
        tile(cw_ref, h, 0, ORG_CW)
        ccw_d[(h, 1)].wait_recv()
        tile(ccw_ref, h, 1, ORG_CCW)

        for d in (cw_d, ccw_d):
            for (hh, pp), desc in d.items():
                if (id(d), hh, pp) not in waited:
                    desc.wait_send()

    return pl.pallas_call(
        body,
        out_shape=jax.ShapeDtypeStruct((N_DEV * m_per, n), jnp.bfloat16),
        in_specs=[
            pl.BlockSpec(memory_space=pltpu.VMEM),
            pl.BlockSpec(memory_space=pltpu.VMEM),
        ],
        out_specs=pl.BlockSpec(memory_space=pltpu.VMEM),
        scratch_shapes=[
            pltpu.VMEM((N_SLOT, N_PIECE, ph, k), jnp.bfloat16),
            pltpu.VMEM((N_SLOT, N_PIECE, ph, k), jnp.bfloat16),
            pltpu.SemaphoreType.DMA((N_SLOT, N_PIECE)),
            pltpu.SemaphoreType.DMA((N_SLOT, N_PIECE)),
            pltpu.SemaphoreType.DMA((N_SLOT, N_PIECE)),
            pltpu.SemaphoreType.DMA((N_SLOT, N_PIECE)),
        ],
        compiler_params=pltpu.CompilerParams(collective_id=0),
    )(A, B)
